# baseline (speedup 1.0000x reference)
import functools

import jax
import jax.numpy as jnp
from jax import lax
from jax.experimental import pallas as pl
from jax.experimental.pallas import tpu as pltpu
from jax.experimental.pallas import tpu_sc as plsc

N = 10000
E = 320000
D1 = 16
D2 = 64
DH = D2 // 2
NC = 2
NS = 16
NW = NC * NS
SUB = 80
S1 = 125
S2 = 250
SH = S2 // 2
NP = 10240
STR = NP // NS

_mesh = plsc.VectorSubcoreMesh(core_axis_name="c", subcore_axis_name="s")
_params = pltpu.CompilerParams(needs_layout_passes=False,
                               use_tc_tiling_on_sc=False)


def _zero_1d(ref, n):
  z16 = jnp.zeros((16,), jnp.float32)
  def body(i, _):
    ref[pl.ds(i * 16, 16)] = z16
    return 0
  lax.fori_loop(0, n // 16, body, 0)


def _zero_rows(ref, nrows, d):
  z16 = jnp.zeros((16,), jnp.float32)
  def body(i, _):
    for k in range(d // 16):
      ref[i, pl.ds(k * 16, 16)] = z16
    return 0
  lax.fori_loop(0, nrows, body, 0)


def _zero_stripe(num_sh, zrows, row0, nrows):
  assert nrows % SUB == 0
  for kk in range(nrows // SUB):
    pltpu.sync_copy(zrows, num_sh.at[pl.ds(row0 + kk * SUB, SUB), :])


def _pipelined_row_pass(hx, src_v, dst_v, num_sh, gbufs, sbufs, sgs, sss,
                        nsteps, d, group_fn, post_issue, post_wait):
  assert nsteps % 2 == 1

  def do_step(j, b, issue_next, wait_prev):
    if issue_next:
      pltpu.async_copy(hx.at[src_v.at[j + 1]], gbufs[1 - b], sgs[1 - b])
    pltpu.make_async_copy(hx.at[src_v.at[j]], gbufs[b], sgs[b]).wait()
    if wait_prev:
      pltpu.make_async_copy(sbufs[b], num_sh.at[dst_v.at[j]], sss[b]).wait()
      post_wait()
    for g in range(SUB // 16):
      sl = pl.ds(g * 16, 16)
      al = group_fn(j, sl)
      for r in range(16):
        row = g * 16 + r
        bc = jnp.full((16,), al[r])
        for k in range(d // 16):
          slk = pl.ds(k * 16, 16)
          sbufs[b][row, slk] = gbufs[b][row, slk] * bc
    post_issue(j)
    pltpu.async_copy(sbufs[b], num_sh.at[dst_v.at[j]], sss[b], add=True)

  pltpu.async_copy(hx.at[src_v.at[0]], gbufs[0], sgs[0])
  do_step(0, 0, True, False)
  do_step(1, 1, True, False)

  def pair(j0, _):
    j = j0 * 2
    do_step(j, 0, True, True)
    do_step(j + 1, 1, True, True)
    return 0

  lax.fori_loop(1, (nsteps - 1) // 2, pair, 0)
  do_step(nsteps - 1, 0, False, True)

  pltpu.make_async_copy(sbufs[1], num_sh.at[dst_v.at[0]], sss[1]).wait()
  pltpu.make_async_copy(sbufs[0], num_sh.at[dst_v.at[0]], sss[0]).wait()
  post_wait()
  post_wait()


def _tc1_body(x_ref, w1_ref, att1_ref, h_ref, aux_ref):
  x = x_ref[...]
  w = w1_ref[...]
  h_ref[...] = jnp.dot(x, w, preferred_element_type=jnp.float32)
  attw = jnp.dot(w, att1_ref[...], preferred_element_type=jnp.float32)
  aux = lax.dot_general(attw, x, (((0,), (1,)), ((), ())),
                        preferred_element_type=jnp.float32)
  aux_ref[...] = aux[:, None, :]


def _tc2_body(num_ref, den_ref, b1_ref, w2_ref, att2_ref, h2_ref, aux2_ref):
  num = num_ref[0, :N, :] + num_ref[1, :N, :]
  den = jnp.sum(den_ref[...], axis=(0, 1))[:N] + 1e-16
  h1 = num / den[:, None] + b1_ref[...]
  h1 = jnp.maximum(h1, 0.0)
  w2 = w2_ref[...]
  h2_ref[...] = jnp.dot(h1, w2, preferred_element_type=jnp.float32)
  attw2 = jnp.dot(w2, att2_ref[...], preferred_element_type=jnp.float32)
  aux2 = lax.dot_general(attw2, h1, (((0,), (1,)), ((), ())),
                         preferred_element_type=jnp.float32)
  aux2_ref[...] = aux2[:, None, :]


def _tc3_body(num2_ref, b2_ref, out_ref):
  h2 = num2_ref[0, :N, :] + num2_ref[1, :N, :] + b2_ref[...]
  m = jnp.max(h2, axis=1, keepdims=True)
  lse = m + jnp.log(jnp.sum(jnp.exp(h2 - m), axis=1, keepdims=True))
  out_ref[...] = h2 - lse


@functools.partial(
    pl.kernel,
    out_type=(
        jax.ShapeDtypeStruct((NC, NP, D1), jnp.float32),
        jax.ShapeDtypeStruct((NW, 1, NP), jnp.float32),
    ),
    mesh=_mesh,
    compiler_params=_params,
    scratch_types=[
        pltpu.VMEM((S1, SUB), jnp.int32),
        pltpu.VMEM((S1, SUB), jnp.int32),
        pltpu.VMEM((N,), jnp.float32),
        pltpu.VMEM((N,), jnp.float32),
        pltpu.VMEM((NP,), jnp.float32),
        pltpu.VMEM((SUB, D1), jnp.float32),
        pltpu.VMEM((SUB, D1), jnp.float32),
        pltpu.VMEM((SUB, D1), jnp.float32),
        pltpu.VMEM((SUB, D1), jnp.float32),
        pltpu.SemaphoreType.DMA,
        pltpu.SemaphoreType.DMA,
        pltpu.SemaphoreType.DMA,
        pltpu.SemaphoreType.DMA,
        pltpu.VMEM_SHARED((NP, D1), jnp.float32),
    ],
)
def _sc_layer1(src_hbm, dst_hbm, h_hbm, aux_hbm, num_out, den_out,
               src_v, dst_v, as_v, ad_v, den_loc, g0, g1, s0, s1,
               sg0, sg1, ss0, ss1, num_sh):
  c = lax.axis_index("c")
  s = lax.axis_index("s")
  w = s * NC + c

  _zero_1d(den_loc, NP)
  _zero_rows(g0, SUB, D1)
  _zero_stripe(num_sh, g0, s * STR, STR)

  pltpu.sync_copy(aux_hbm.at[0, 0], as_v)
  pltpu.sync_copy(aux_hbm.at[1, 0], ad_v)
  pltpu.sync_copy(src_hbm.at[w], src_v)
  pltpu.sync_copy(dst_hbm.at[w], dst_v)
  plsc.subcore_barrier()

  def group_fn(j, sl):
    s16 = src_v[j, sl]
    d16 = dst_v[j, sl]
    e = plsc.load_gather(as_v, [s16]) + plsc.load_gather(ad_v, [d16])
    e = jnp.maximum(e, 0.2 * e)
    ex = jnp.exp(e)
    plsc.addupdate_scatter(den_loc, [d16], ex)
    return ex

  def no_issue(j):
    pass

  def no_wait():
    pass

  _pipelined_row_pass(h_hbm, src_v, dst_v, num_sh, [g0, g1], [s0, s1],
                      [sg0, sg1], [ss0, ss1], S1, D1,
                      group_fn, no_issue, no_wait)

  pltpu.sync_copy(den_loc, den_out.at[w, 0])
  plsc.subcore_barrier()
  pltpu.sync_copy(num_sh.at[pl.ds(s * STR, STR), :],
                  num_out.at[c, pl.ds(s * STR, STR), :])


CH = 8
NCHK = 31


@functools.partial(
    pl.kernel,
    out_type=(
        jax.ShapeDtypeStruct((NC, NP, D2), jnp.float32),
        jax.ShapeDtypeStruct((NS, NC, SH, SUB), jnp.float32),
    ),
    mesh=_mesh,
    compiler_params=_params,
    scratch_types=[
        pltpu.VMEM((CH, SUB), jnp.int32),
        pltpu.VMEM((CH, SUB), jnp.int32),
        pltpu.VMEM((CH, SUB), jnp.int32),
        pltpu.VMEM((CH, SUB), jnp.int32),
        pltpu.VMEM((2, SUB), jnp.int32),
        pltpu.VMEM((2, SUB), jnp.int32),
        pltpu.VMEM((SH, SUB), jnp.int32),
        pltpu.VMEM((SH, SUB), jnp.int32),
        pltpu.VMEM((N,), jnp.float32),
        pltpu.VMEM((N,), jnp.float32),
        pltpu.VMEM((NP // D2, D2), jnp.float32),
        pltpu.VMEM((NP // D2 // NS, D2), jnp.float32),
        pltpu.VMEM((NP // D2 // NS, D2), jnp.float32),
        pltpu.VMEM((4, SUB), jnp.float32),
        pltpu.VMEM((SUB, D2), jnp.float32),
        pltpu.VMEM((SUB, D2), jnp.float32),
        pltpu.VMEM((SUB, D2), jnp.float32),
        pltpu.VMEM((SUB, D2), jnp.float32),
        pltpu.SemaphoreType.DMA,
        pltpu.SemaphoreType.DMA,
        pltpu.SemaphoreType.DMA,
        pltpu.SemaphoreType.DMA,
        pltpu.SemaphoreType.DMA,
        pltpu.SemaphoreType.DMA,
        pltpu.SemaphoreType.DMA,
        pltpu.VMEM_SHARED((NP // D2, D2), jnp.float32),
        pltpu.VMEM_SHARED((NP, D2), jnp.float32),
    ],
)
def _sc_layer2(srcsc_hbm, dstsc_hbm, h2_hbm, aux_hbm,
               num_out, alpha_out, rs0, rs1, rd0, rd1, tsrc_v, tdst_v,
               srow_v, drow_v, as_v, ad_v, den_v, tmp_v, acc_v, alr_v,
               g0, g1, s0, s1, sg0, sg1, ss0, ss1, sal, sr0, sr1,
               den_sh, num_sh):
  c = lax.axis_index("c")
  s = lax.axis_index("s")
  DR = NP // D2
  DT = DR // NS

  _zero_rows(den_v, DR, D2)
  _zero_rows(g0, SUB, D2)

  pltpu.sync_copy(aux_hbm.at[0, 0], as_v)
  pltpu.sync_copy(aux_hbm.at[1, 0], ad_v)
  pltpu.sync_copy(srcsc_hbm.at[s, pl.ds(c * SH, SH)], srow_v)
  pltpu.sync_copy(dstsc_hbm.at[s, pl.ds(c * SH, SH)], drow_v)
  pltpu.sync_copy(srcsc_hbm.at[s, pl.ds(NCHK * CH, 2)], tsrc_v)
  pltpu.sync_copy(dstsc_hbm.at[s, pl.ds(NCHK * CH, 2)], tdst_v)

  rs = [rs0, rs1]
  rd = [rd0, rd1]
  srr = [sr0, sr1]

  def stage_chunk(q, b):
    pltpu.async_copy(srcsc_hbm.at[s, pl.ds(q * CH, CH)], rs[b], srr[b])
    pltpu.async_copy(dstsc_hbm.at[s, pl.ds(q * CH, CH)], rd[b], srr[b])

  def wait_chunk(b):
    pltpu.make_async_copy(srcsc_hbm.at[s, pl.ds(0, CH)], rs[b], srr[b]).wait()
    pltpu.make_async_copy(dstsc_hbm.at[s, pl.ds(0, CH)], rd[b], srr[b]).wait()

  def sstep(sref, dref, bb):
    for g in range(SUB // 16):
      sl = pl.ds(g * 16, 16)
      s16 = sref[bb, sl]
      d16 = dref[bb, sl]
      e = plsc.load_gather(as_v, [s16]) + plsc.load_gather(ad_v, [d16])
      e = jnp.maximum(e, 0.2 * e)
      plsc.addupdate_scatter(den_v, [d16 >> 6, d16 & 63], jnp.exp(e))

  stage_chunk(0, 0)
  stage_chunk(1, 1)
  wait_chunk(0)
  for bb in range(CH):
    sstep(rs0, rd0, bb)
  stage_chunk(2, 0)

  def cpair(p, _):
    q1 = 2 * p + 1
    wait_chunk(1)
    for bb in range(CH):
      sstep(rs1, rd1, bb)
    pl.when(q1 + 2 < NCHK)(lambda: stage_chunk(q1 + 2, 1))
    q2 = 2 * p + 2
    wait_chunk(0)
    for bb in range(CH):
      sstep(rs0, rd0, bb)
    pl.when(q2 + 2 < NCHK)(lambda: stage_chunk(q2 + 2, 0))
    return 0

  lax.fori_loop(0, (NCHK - 1) // 2, cpair, 0)
  for bb in range(2):
    sstep(tsrc_v, tdst_v, bb)

  pltpu.sync_copy(den_v, num_sh.at[pl.ds(s * DR, DR), :])
  plsc.subcore_barrier()
  _zero_rows(acc_v, DT, D2)

  def red(t, _):
    pltpu.sync_copy(num_sh.at[pl.ds(t * DR + s * DT, DT), :], tmp_v)
    def addq(i, _):
      for k in range(D2 // 16):
        slk = pl.ds(k * 16, 16)
        acc_v[i, slk] = acc_v[i, slk] + tmp_v[i, slk]
      return 0
    lax.fori_loop(0, DT, addq, 0)
    return 0

  lax.fori_loop(0, NS, red, 0)
  pltpu.sync_copy(acc_v, den_sh.at[pl.ds(s * DT, DT), :])
  plsc.subcore_barrier()
  pltpu.sync_copy(den_sh, den_v)
  _zero_stripe(num_sh, g0, s * STR, STR)
  plsc.subcore_barrier()

  def group_a(j, sl):
    s16 = srow_v[j, sl]
    d16 = drow_v[j, sl]
    e = plsc.load_gather(as_v, [s16]) + plsc.load_gather(ad_v, [d16])
    e = jnp.maximum(e, 0.2 * e)
    den16 = plsc.load_gather(den_v, [d16 >> 6, d16 & 63]) + 1e-16
    al = jnp.exp(e) / den16
    alr_v[lax.rem(j, 4), sl] = al
    return al

  def alpha_issue(j):
    pltpu.async_copy(alr_v.at[lax.rem(j, 4)], alpha_out.at[s, c, j], sal)

  def alpha_wait():
    pltpu.make_async_copy(alr_v.at[0], alpha_out.at[s, c, 0], sal).wait()

  _pipelined_row_pass(h2_hbm, srow_v, drow_v, num_sh, [g0, g1], [s0, s1],
                      [sg0, sg1], [ss0, ss1], SH, D2,
                      group_a, alpha_issue, alpha_wait)
  plsc.subcore_barrier()
  pltpu.sync_copy(num_sh.at[pl.ds(s * STR, STR), :],
                  num_out.at[c, pl.ds(s * STR, STR), :])


def kernel(x, edge_index, edge_weight, W1, att_src1, att_dst1, bias1,
           W2, att_src2, att_dst2, bias2):
  del edge_weight
  src = edge_index[0].astype(jnp.int32)
  dst = edge_index[1].astype(jnp.int32)
  src1 = src.reshape(NW, S1, SUB)
  dst1 = dst.reshape(NW, S1, SUB)
  src2 = src.reshape(NS, S2, SUB)
  dst2 = dst.reshape(NS, S2, SUB)

  att1 = jnp.stack([att_src1, att_dst1], axis=1)
  att2 = jnp.stack([att_src2, att_dst2], axis=1)

  h1, aux1 = pl.pallas_call(
      _tc1_body,
      out_shape=(jax.ShapeDtypeStruct((N, D1), jnp.float32),
                 jax.ShapeDtypeStruct((2, 1, N), jnp.float32)),
  )(x, W1, att1)

  num1p, den1p = _sc_layer1(src1, dst1, h1, aux1)

  h2, aux2 = pl.pallas_call(
      _tc2_body,
      out_shape=(jax.ShapeDtypeStruct((N, D2), jnp.float32),
                 jax.ShapeDtypeStruct((2, 1, N), jnp.float32)),
  )(num1p, den1p, bias1.reshape(1, D1), W2, att2)

  num2p, alpha4 = _sc_layer2(src2, dst2, h2, aux2)

  logp = pl.pallas_call(
      _tc3_body,
      out_shape=jax.ShapeDtypeStruct((N, D2), jnp.float32),
  )(num2p, bias2.reshape(1, D2))

  alpha = alpha4.reshape(E, 1)

  return (logp, edge_index, alpha)


# speedup vs baseline: 1.0254x; 1.0254x over previous
import functools

import jax
import jax.numpy as jnp
from jax import lax
from jax.experimental import pallas as pl
from jax.experimental.pallas import tpu as pltpu
from jax.experimental.pallas import tpu_sc as plsc

N = 10000
E = 320000
D1 = 16
D2 = 64
DH = D2 // 2
NC = 2
NS = 16
NW = NC * NS
SUB = 80
S1 = 125
S2 = 250
SH = S2 // 2
NP = 10240
STR = NP // NS

_mesh = plsc.VectorSubcoreMesh(core_axis_name="c", subcore_axis_name="s")
_params = pltpu.CompilerParams(needs_layout_passes=False,
                               use_tc_tiling_on_sc=False)


def _zero_1d(ref, n):
  z16 = jnp.zeros((16,), jnp.float32)
  def body(i, _):
    ref[pl.ds(i * 16, 16)] = z16
    return 0
  lax.fori_loop(0, n // 16, body, 0)


def _zero_rows(ref, nrows, d):
  z16 = jnp.zeros((16,), jnp.float32)
  def body(i, _):
    for k in range(d // 16):
      ref[i, pl.ds(k * 16, 16)] = z16
    return 0
  lax.fori_loop(0, nrows, body, 0)


def _zero_stripe(num_sh, zrows, row0, nrows):
  assert nrows % SUB == 0
  for kk in range(nrows // SUB):
    pltpu.sync_copy(zrows, num_sh.at[pl.ds(row0 + kk * SUB, SUB), :])


def _pipelined_row_pass(hx, src_v, dst_v, num_sh, gbufs, sbufs, sgs, sss,
                        nsteps, d, group_fn, post_issue, post_wait):
  assert nsteps % 2 == 1

  def do_step(j, b, issue_next, wait_prev):
    if issue_next:
      pltpu.async_copy(hx.at[src_v.at[j + 1]], gbufs[1 - b], sgs[1 - b])
    pltpu.make_async_copy(hx.at[src_v.at[j]], gbufs[b], sgs[b]).wait()
    if wait_prev:
      pltpu.make_async_copy(sbufs[b], num_sh.at[dst_v.at[j]], sss[b]).wait()
      post_wait()
    for g in range(SUB // 16):
      sl = pl.ds(g * 16, 16)
      al = group_fn(j, sl)
      for r in range(16):
        row = g * 16 + r
        bc = jnp.full((16,), al[r])
        for k in range(d // 16):
          slk = pl.ds(k * 16, 16)
          sbufs[b][row, slk] = gbufs[b][row, slk] * bc
    post_issue(j)
    pltpu.async_copy(sbufs[b], num_sh.at[dst_v.at[j]], sss[b], add=True)

  pltpu.async_copy(hx.at[src_v.at[0]], gbufs[0], sgs[0])
  do_step(0, 0, True, False)
  do_step(1, 1, True, False)

  def pair(j0, _):
    j = j0 * 2
    do_step(j, 0, True, True)
    do_step(j + 1, 1, True, True)
    return 0

  lax.fori_loop(1, (nsteps - 1) // 2, pair, 0)
  do_step(nsteps - 1, 0, False, True)

  pltpu.make_async_copy(sbufs[1], num_sh.at[dst_v.at[0]], sss[1]).wait()
  pltpu.make_async_copy(sbufs[0], num_sh.at[dst_v.at[0]], sss[0]).wait()
  post_wait()
  post_wait()


def _tc1_body(x_ref, w1_ref, att1_ref, h_ref, aux_ref):
  x = x_ref[...]
  w = w1_ref[...]
  h_ref[...] = jnp.dot(x, w, preferred_element_type=jnp.float32)
  attw = jnp.dot(w, att1_ref[...], preferred_element_type=jnp.float32)
  aux = lax.dot_general(attw, x, (((0,), (1,)), ((), ())),
                        preferred_element_type=jnp.float32)
  aux_ref[...] = aux[:, None, :]


def _tc2_body(num_ref, den_ref, b1_ref, w2_ref, att2_ref, h2_ref, aux2_ref):
  num = num_ref[0, :N, :] + num_ref[1, :N, :]
  den = jnp.sum(den_ref[...], axis=(0, 1))[:N] + 1e-16
  h1 = num / den[:, None] + b1_ref[...]
  h1 = jnp.maximum(h1, 0.0)
  w2 = w2_ref[...]
  h2_ref[...] = jnp.dot(h1, w2, preferred_element_type=jnp.float32)
  attw2 = jnp.dot(w2, att2_ref[...], preferred_element_type=jnp.float32)
  aux2 = lax.dot_general(attw2, h1, (((0,), (1,)), ((), ())),
                         preferred_element_type=jnp.float32)
  aux2_ref[...] = aux2[:, None, :]


def _tc3_body(num2_ref, b2_ref, out_ref):
  h2 = num2_ref[0, :N, :] + num2_ref[1, :N, :] + b2_ref[...]
  m = jnp.max(h2, axis=1, keepdims=True)
  lse = m + jnp.log(jnp.sum(jnp.exp(h2 - m), axis=1, keepdims=True))
  out_ref[...] = h2 - lse


@functools.partial(
    pl.kernel,
    out_type=(
        jax.ShapeDtypeStruct((NC, NP, D1), jnp.float32),
        jax.ShapeDtypeStruct((NW, 1, NP), jnp.float32),
    ),
    mesh=_mesh,
    compiler_params=_params,
    scratch_types=[
        pltpu.VMEM((S1, SUB), jnp.int32),
        pltpu.VMEM((S1, SUB), jnp.int32),
        pltpu.VMEM((N,), jnp.float32),
        pltpu.VMEM((N,), jnp.float32),
        pltpu.VMEM((NP,), jnp.float32),
        pltpu.VMEM((SUB, D1), jnp.float32),
        pltpu.VMEM((SUB, D1), jnp.float32),
        pltpu.VMEM((SUB, D1), jnp.float32),
        pltpu.VMEM((SUB, D1), jnp.float32),
        pltpu.SemaphoreType.DMA,
        pltpu.SemaphoreType.DMA,
        pltpu.SemaphoreType.DMA,
        pltpu.SemaphoreType.DMA,
        pltpu.VMEM_SHARED((NP, D1), jnp.float32),
    ],
)
def _sc_layer1(src_hbm, dst_hbm, h_hbm, aux_hbm, num_out, den_out,
               src_v, dst_v, as_v, ad_v, den_loc, g0, g1, s0, s1,
               sg0, sg1, ss0, ss1, num_sh):
  c = lax.axis_index("c")
  s = lax.axis_index("s")
  w = s * NC + c

  _zero_1d(den_loc, NP)
  _zero_rows(g0, SUB, D1)
  _zero_stripe(num_sh, g0, s * STR, STR)

  pltpu.sync_copy(aux_hbm.at[0, 0], as_v)
  pltpu.sync_copy(aux_hbm.at[1, 0], ad_v)
  pltpu.sync_copy(src_hbm.at[w], src_v)
  pltpu.sync_copy(dst_hbm.at[w], dst_v)
  plsc.subcore_barrier()

  def group_fn(j, sl):
    s16 = src_v[j, sl]
    d16 = dst_v[j, sl]
    e = plsc.load_gather(as_v, [s16]) + plsc.load_gather(ad_v, [d16])
    e = jnp.maximum(e, 0.2 * e)
    ex = jnp.exp(e)
    plsc.addupdate_scatter(den_loc, [d16], ex)
    return ex

  def no_issue(j):
    pass

  def no_wait():
    pass

  _pipelined_row_pass(h_hbm, src_v, dst_v, num_sh, [g0, g1], [s0, s1],
                      [sg0, sg1], [ss0, ss1], S1, D1,
                      group_fn, no_issue, no_wait)

  pltpu.sync_copy(den_loc, den_out.at[w, 0])
  plsc.subcore_barrier()
  pltpu.sync_copy(num_sh.at[pl.ds(s * STR, STR), :],
                  num_out.at[c, pl.ds(s * STR, STR), :])


CH = 8
NCHK = 31


@functools.partial(
    pl.kernel,
    out_type=(
        jax.ShapeDtypeStruct((NC, NP, D2), jnp.float32),
        jax.ShapeDtypeStruct((NS, NC, SH, SUB), jnp.float32),
    ),
    mesh=_mesh,
    compiler_params=_params,
    scratch_types=[
        pltpu.VMEM((CH, SUB), jnp.int32),
        pltpu.VMEM((CH, SUB), jnp.int32),
        pltpu.VMEM((CH, SUB), jnp.int32),
        pltpu.VMEM((CH, SUB), jnp.int32),
        pltpu.VMEM((2, SUB), jnp.int32),
        pltpu.VMEM((2, SUB), jnp.int32),
        pltpu.VMEM((SH, SUB), jnp.int32),
        pltpu.VMEM((SH, SUB), jnp.int32),
        pltpu.VMEM((N,), jnp.float32),
        pltpu.VMEM((N,), jnp.float32),
        pltpu.VMEM((N,), jnp.float32),
        pltpu.VMEM((4, SUB), jnp.float32),
        pltpu.VMEM((4, SUB), jnp.float32),
        pltpu.VMEM((STR,), jnp.float32),
        pltpu.VMEM((SUB, D2), jnp.float32),
        pltpu.VMEM((SUB, D2), jnp.float32),
        pltpu.VMEM((SUB, D2), jnp.float32),
        pltpu.VMEM((SUB, D2), jnp.float32),
        pltpu.SemaphoreType.DMA,
        pltpu.SemaphoreType.DMA,
        pltpu.SemaphoreType.DMA,
        pltpu.SemaphoreType.DMA,
        pltpu.SemaphoreType.DMA,
        pltpu.SemaphoreType.DMA,
        pltpu.SemaphoreType.DMA,
        pltpu.SemaphoreType.DMA,
        pltpu.VMEM_SHARED((NP,), jnp.float32),
        pltpu.VMEM_SHARED((NP, D2), jnp.float32),
    ],
)
def _sc_layer2(srcsc_hbm, dstsc_hbm, h2_hbm, aux_hbm,
               num_out, alpha_out, rs0, rs1, rd0, rd1, tsrc_v, tdst_v,
               srow_v, drow_v, as_v, ad_v, denf_v, exr_v, alr_v, zstr_v,
               g0, g1, s0, s1, sg0, sg1, ss0, ss1, sd, sal, sr0, sr1,
               den_sh, num_sh):
  c = lax.axis_index("c")
  s = lax.axis_index("s")

  _zero_1d(zstr_v, STR)
  _zero_rows(g0, SUB, D2)
  _zero_stripe(num_sh, g0, s * STR, STR)
  pltpu.sync_copy(zstr_v, den_sh.at[pl.ds(s * STR, STR)])

  pltpu.sync_copy(aux_hbm.at[0, 0], as_v)
  pltpu.sync_copy(aux_hbm.at[1, 0], ad_v)
  pltpu.sync_copy(srcsc_hbm.at[s, pl.ds(c * SH, SH)], srow_v)
  pltpu.sync_copy(dstsc_hbm.at[s, pl.ds(c * SH, SH)], drow_v)
  pltpu.sync_copy(srcsc_hbm.at[s, pl.ds(NCHK * CH, 2)], tsrc_v)
  pltpu.sync_copy(dstsc_hbm.at[s, pl.ds(NCHK * CH, 2)], tdst_v)
  plsc.subcore_barrier()

  rs = [rs0, rs1]
  rd = [rd0, rd1]
  srr = [sr0, sr1]

  def stage_chunk(q, b):
    pltpu.async_copy(srcsc_hbm.at[s, pl.ds(q * CH, CH)], rs[b], srr[b])
    pltpu.async_copy(dstsc_hbm.at[s, pl.ds(q * CH, CH)], rd[b], srr[b])

  def wait_chunk(b):
    pltpu.make_async_copy(srcsc_hbm.at[s, pl.ds(0, CH)], rs[b], srr[b]).wait()
    pltpu.make_async_copy(dstsc_hbm.at[s, pl.ds(0, CH)], rd[b], srr[b]).wait()

  def den_wait():
    pltpu.make_async_copy(exr_v.at[0], den_sh.at[rd0.at[0]], sd).wait()

  def sstep(sref, dref, bb, dwait):
    jm = bb % 4
    for g in range(SUB // 16):
      sl = pl.ds(g * 16, 16)
      s16 = sref[bb, sl]
      d16 = dref[bb, sl]
      e = plsc.load_gather(as_v, [s16]) + plsc.load_gather(ad_v, [d16])
      e = jnp.maximum(e, 0.2 * e)
      exr_v[jm, sl] = jnp.exp(e)
    if dwait:
      den_wait()
    pltpu.async_copy(exr_v.at[jm], den_sh.at[dref.at[bb]], sd, add=True)

  stage_chunk(0, 0)
  stage_chunk(1, 1)
  wait_chunk(0)
  for bb in range(CH):
    sstep(rs0, rd0, bb, bb >= 2)
  stage_chunk(2, 0)

  def cpair(p, _):
    q1 = 2 * p + 1
    wait_chunk(1)
    for bb in range(CH):
      sstep(rs1, rd1, bb, True)
    pl.when(q1 + 2 < NCHK)(lambda: stage_chunk(q1 + 2, 1))
    q2 = 2 * p + 2
    wait_chunk(0)
    for bb in range(CH):
      sstep(rs0, rd0, bb, True)
    pl.when(q2 + 2 < NCHK)(lambda: stage_chunk(q2 + 2, 0))
    return 0

  lax.fori_loop(0, (NCHK - 1) // 2, cpair, 0)
  for bb in range(2):
    sstep(tsrc_v, tdst_v, bb, True)
  den_wait()
  den_wait()

  plsc.subcore_barrier()
  pltpu.sync_copy(den_sh.at[pl.ds(0, N)], denf_v)

  def group_a(j, sl):
    s16 = srow_v[j, sl]
    d16 = drow_v[j, sl]
    e = plsc.load_gather(as_v, [s16]) + plsc.load_gather(ad_v, [d16])
    e = jnp.maximum(e, 0.2 * e)
    den16 = plsc.load_gather(denf_v, [d16]) + 1e-16
    al = jnp.exp(e) / den16
    alr_v[lax.rem(j, 4), sl] = al
    return al

  def alpha_issue(j):
    pltpu.async_copy(alr_v.at[lax.rem(j, 4)], alpha_out.at[s, c, j], sal)

  def alpha_wait():
    pltpu.make_async_copy(alr_v.at[0], alpha_out.at[s, c, 0], sal).wait()

  _pipelined_row_pass(h2_hbm, srow_v, drow_v, num_sh, [g0, g1], [s0, s1],
                      [sg0, sg1], [ss0, ss1], SH, D2,
                      group_a, alpha_issue, alpha_wait)
  plsc.subcore_barrier()
  pltpu.sync_copy(num_sh.at[pl.ds(s * STR, STR), :],
                  num_out.at[c, pl.ds(s * STR, STR), :])


def kernel(x, edge_index, edge_weight, W1, att_src1, att_dst1, bias1,
           W2, att_src2, att_dst2, bias2):
  del edge_weight
  src = edge_index[0].astype(jnp.int32)
  dst = edge_index[1].astype(jnp.int32)
  src1 = src.reshape(NW, S1, SUB)
  dst1 = dst.reshape(NW, S1, SUB)
  src2 = src.reshape(NS, S2, SUB)
  dst2 = dst.reshape(NS, S2, SUB)

  att1 = jnp.stack([att_src1, att_dst1], axis=1)
  att2 = jnp.stack([att_src2, att_dst2], axis=1)

  h1, aux1 = pl.pallas_call(
      _tc1_body,
      out_shape=(jax.ShapeDtypeStruct((N, D1), jnp.float32),
                 jax.ShapeDtypeStruct((2, 1, N), jnp.float32)),
  )(x, W1, att1)

  num1p, den1p = _sc_layer1(src1, dst1, h1, aux1)

  h2, aux2 = pl.pallas_call(
      _tc2_body,
      out_shape=(jax.ShapeDtypeStruct((N, D2), jnp.float32),
                 jax.ShapeDtypeStruct((2, 1, N), jnp.float32)),
  )(num1p, den1p, bias1.reshape(1, D1), W2, att2)

  num2p, alpha4 = _sc_layer2(src2, dst2, h2, aux2)

  logp = pl.pallas_call(
      _tc3_body,
      out_shape=jax.ShapeDtypeStruct((N, D2), jnp.float32),
  )(num2p, bias2.reshape(1, D2))

  alpha = alpha4.reshape(E, 1)

  return (logp, edge_index, alpha)


# speedup vs baseline: 1.1579x; 1.1292x over previous
import functools

import jax
import jax.numpy as jnp
from jax import lax
from jax.experimental import pallas as pl
from jax.experimental.pallas import tpu as pltpu
from jax.experimental.pallas import tpu_sc as plsc

N = 10000
E = 320000
D1 = 16
D2 = 64
DH = D2 // 2
NC = 2
NS = 16
NW = NC * NS
SUB = 80
S1 = 125
S2 = 250
SH = S2 // 2
NP = 10240
STR = NP // NS

_mesh = plsc.VectorSubcoreMesh(core_axis_name="c", subcore_axis_name="s")
_params = pltpu.CompilerParams(needs_layout_passes=False,
                               use_tc_tiling_on_sc=False)


def _zero_1d(ref, n):
  z16 = jnp.zeros((16,), jnp.float32)
  def body(i, _):
    ref[pl.ds(i * 16, 16)] = z16
    return 0
  lax.fori_loop(0, n // 16, body, 0)


def _zero_rows(ref, nrows, d):
  z16 = jnp.zeros((16,), jnp.float32)
  def body(i, _):
    for k in range(d // 16):
      ref[i, pl.ds(k * 16, 16)] = z16
    return 0
  lax.fori_loop(0, nrows, body, 0)


def _zero_stripe(num_sh, zrows, row0, nrows):
  assert nrows % SUB == 0
  for kk in range(nrows // SUB):
    pltpu.sync_copy(zrows, num_sh.at[pl.ds(row0 + kk * SUB, SUB), :])


def _pipelined_row_pass(hx, src_v, dst_v, num_sh, gbufs, sbufs, sgs, sss,
                        nsteps, d, group_fn, post_issue, post_wait):
  assert nsteps % 2 == 1

  def do_step(j, b, issue_next, wait_prev):
    if issue_next:
      pltpu.async_copy(hx.at[src_v.at[j + 1]], gbufs[1 - b], sgs[1 - b])
    pltpu.make_async_copy(hx.at[src_v.at[j]], gbufs[b], sgs[b]).wait()
    if wait_prev:
      pltpu.make_async_copy(sbufs[b], num_sh.at[dst_v.at[j]], sss[b]).wait()
      post_wait()
    for g in range(SUB // 16):
      sl = pl.ds(g * 16, 16)
      al = group_fn(j, sl)
      for r in range(16):
        row = g * 16 + r
        bc = jnp.full((16,), al[r])
        for k in range(d // 16):
          slk = pl.ds(k * 16, 16)
          sbufs[b][row, slk] = gbufs[b][row, slk] * bc
    post_issue(j)
    pltpu.async_copy(sbufs[b], num_sh.at[dst_v.at[j]], sss[b], add=True)

  pltpu.async_copy(hx.at[src_v.at[0]], gbufs[0], sgs[0])
  do_step(0, 0, True, False)
  do_step(1, 1, True, False)

  def pair(j0, _):
    j = j0 * 2
    do_step(j, 0, True, True)
    do_step(j + 1, 1, True, True)
    return 0

  lax.fori_loop(1, (nsteps - 1) // 2, pair, 0)
  do_step(nsteps - 1, 0, False, True)

  pltpu.make_async_copy(sbufs[1], num_sh.at[dst_v.at[0]], sss[1]).wait()
  pltpu.make_async_copy(sbufs[0], num_sh.at[dst_v.at[0]], sss[0]).wait()
  post_wait()
  post_wait()


def _tc1_body(x_ref, w1_ref, att1_ref, h_ref, aux_ref):
  x = x_ref[...]
  w = w1_ref[...]
  h_ref[...] = jnp.dot(x, w, preferred_element_type=jnp.float32)
  attw = jnp.dot(w, att1_ref[...], preferred_element_type=jnp.float32)
  aux = lax.dot_general(attw, x, (((0,), (1,)), ((), ())),
                        preferred_element_type=jnp.float32)
  aux_ref[...] = aux[:, None, :]


def _tc2_body(num_ref, den_ref, b1_ref, w2_ref, att2_ref, h2_ref, aux2_ref):
  num = num_ref[0, :N, :] + num_ref[1, :N, :]
  den = jnp.sum(den_ref[...], axis=(0, 1))[:N] + 1e-16
  h1 = num / den[:, None] + b1_ref[...]
  h1 = jnp.maximum(h1, 0.0)
  w2 = w2_ref[...]
  h2_ref[...] = jnp.dot(h1, w2, preferred_element_type=jnp.float32)
  attw2 = jnp.dot(w2, att2_ref[...], preferred_element_type=jnp.float32)
  aux2 = lax.dot_general(attw2, h1, (((0,), (1,)), ((), ())),
                         preferred_element_type=jnp.float32)
  aux2_ref[...] = aux2[:, None, :]


def _tc3_body(num2_ref, b2_ref, out_ref):
  h2 = num2_ref[0, :N, :] + num2_ref[1, :N, :] + b2_ref[...]
  m = jnp.max(h2, axis=1, keepdims=True)
  lse = m + jnp.log(jnp.sum(jnp.exp(h2 - m), axis=1, keepdims=True))
  out_ref[...] = h2 - lse


@functools.partial(
    pl.kernel,
    out_type=(
        jax.ShapeDtypeStruct((NC, NP, D1), jnp.float32),
        jax.ShapeDtypeStruct((NW, 1, NP), jnp.float32),
    ),
    mesh=_mesh,
    compiler_params=_params,
    scratch_types=[
        pltpu.VMEM((S1, SUB), jnp.int32),
        pltpu.VMEM((S1, SUB), jnp.int32),
        pltpu.VMEM((N,), jnp.float32),
        pltpu.VMEM((N,), jnp.float32),
        pltpu.VMEM((NP,), jnp.float32),
        pltpu.VMEM((SUB, D1), jnp.float32),
        pltpu.VMEM((SUB, D1), jnp.float32),
        pltpu.VMEM((SUB, D1), jnp.float32),
        pltpu.VMEM((SUB, D1), jnp.float32),
        pltpu.SemaphoreType.DMA,
        pltpu.SemaphoreType.DMA,
        pltpu.SemaphoreType.DMA,
        pltpu.SemaphoreType.DMA,
        pltpu.VMEM_SHARED((NP, D1), jnp.float32),
        pltpu.VMEM_SHARED((N, D1), jnp.float32),
    ],
)
def _sc_layer1(src_hbm, dst_hbm, h_hbm, aux_hbm, num_out, den_out,
               src_v, dst_v, as_v, ad_v, den_loc, g0, g1, s0, s1,
               sg0, sg1, ss0, ss1, num_sh, h_sh):
  c = lax.axis_index("c")
  s = lax.axis_index("s")
  w = s * NC + c

  _zero_1d(den_loc, NP)
  _zero_rows(g0, SUB, D1)
  _zero_stripe(num_sh, g0, s * STR, STR)

  nh = N // NS
  pltpu.sync_copy(h_hbm.at[pl.ds(s * nh, nh), :], h_sh.at[pl.ds(s * nh, nh), :])

  pltpu.sync_copy(aux_hbm.at[0, 0], as_v)
  pltpu.sync_copy(aux_hbm.at[1, 0], ad_v)
  pltpu.sync_copy(src_hbm.at[w], src_v)
  pltpu.sync_copy(dst_hbm.at[w], dst_v)
  plsc.subcore_barrier()

  def group_fn(j, sl):
    s16 = src_v[j, sl]
    d16 = dst_v[j, sl]
    e = plsc.load_gather(as_v, [s16]) + plsc.load_gather(ad_v, [d16])
    e = jnp.maximum(e, 0.2 * e)
    ex = jnp.exp(e)
    plsc.addupdate_scatter(den_loc, [d16], ex)
    return ex

  def no_issue(j):
    pass

  def no_wait():
    pass

  _pipelined_row_pass(h_sh, src_v, dst_v, num_sh, [g0, g1], [s0, s1],
                      [sg0, sg1], [ss0, ss1], S1, D1,
                      group_fn, no_issue, no_wait)

  pltpu.sync_copy(den_loc, den_out.at[w, 0])
  plsc.subcore_barrier()
  pltpu.sync_copy(num_sh.at[pl.ds(s * STR, STR), :],
                  num_out.at[c, pl.ds(s * STR, STR), :])


CH = 8
NCHK = 31


@functools.partial(
    pl.kernel,
    out_type=(
        jax.ShapeDtypeStruct((NC, NP, D2), jnp.float32),
        jax.ShapeDtypeStruct((NS, NC, SH, SUB), jnp.float32),
    ),
    mesh=_mesh,
    compiler_params=_params,
    scratch_types=[
        pltpu.VMEM((CH, SUB), jnp.int32),
        pltpu.VMEM((CH, SUB), jnp.int32),
        pltpu.VMEM((CH, SUB), jnp.int32),
        pltpu.VMEM((CH, SUB), jnp.int32),
        pltpu.VMEM((2, SUB), jnp.int32),
        pltpu.VMEM((2, SUB), jnp.int32),
        pltpu.VMEM((SH, SUB), jnp.int32),
        pltpu.VMEM((SH, SUB), jnp.int32),
        pltpu.VMEM((N,), jnp.float32),
        pltpu.VMEM((N,), jnp.float32),
        pltpu.VMEM((N,), jnp.float32),
        pltpu.VMEM((4, SUB), jnp.float32),
        pltpu.VMEM((4, SUB), jnp.float32),
        pltpu.VMEM((STR,), jnp.float32),
        pltpu.VMEM((SUB, D2), jnp.float32),
        pltpu.VMEM((SUB, D2), jnp.float32),
        pltpu.VMEM((SUB, D2), jnp.float32),
        pltpu.VMEM((SUB, D2), jnp.float32),
        pltpu.SemaphoreType.DMA,
        pltpu.SemaphoreType.DMA,
        pltpu.SemaphoreType.DMA,
        pltpu.SemaphoreType.DMA,
        pltpu.SemaphoreType.DMA,
        pltpu.SemaphoreType.DMA,
        pltpu.SemaphoreType.DMA,
        pltpu.SemaphoreType.DMA,
        pltpu.VMEM_SHARED((NP,), jnp.float32),
        pltpu.VMEM_SHARED((NP, D2), jnp.float32),
    ],
)
def _sc_layer2(srcsc_hbm, dstsc_hbm, h2_hbm, aux_hbm,
               num_out, alpha_out, rs0, rs1, rd0, rd1, tsrc_v, tdst_v,
               srow_v, drow_v, as_v, ad_v, denf_v, exr_v, alr_v, zstr_v,
               g0, g1, s0, s1, sg0, sg1, ss0, ss1, sd, sal, sr0, sr1,
               den_sh, num_sh):
  c = lax.axis_index("c")
  s = lax.axis_index("s")

  _zero_1d(zstr_v, STR)
  _zero_rows(g0, SUB, D2)
  _zero_stripe(num_sh, g0, s * STR, STR)
  pltpu.sync_copy(zstr_v, den_sh.at[pl.ds(s * STR, STR)])

  pltpu.sync_copy(aux_hbm.at[0, 0], as_v)
  pltpu.sync_copy(aux_hbm.at[1, 0], ad_v)
  pltpu.sync_copy(srcsc_hbm.at[s, pl.ds(c * SH, SH)], srow_v)
  pltpu.sync_copy(dstsc_hbm.at[s, pl.ds(c * SH, SH)], drow_v)
  pltpu.sync_copy(srcsc_hbm.at[s, pl.ds(NCHK * CH, 2)], tsrc_v)
  pltpu.sync_copy(dstsc_hbm.at[s, pl.ds(NCHK * CH, 2)], tdst_v)
  plsc.subcore_barrier()

  rs = [rs0, rs1]
  rd = [rd0, rd1]
  srr = [sr0, sr1]

  def stage_chunk(q, b):
    pltpu.async_copy(srcsc_hbm.at[s, pl.ds(q * CH, CH)], rs[b], srr[b])
    pltpu.async_copy(dstsc_hbm.at[s, pl.ds(q * CH, CH)], rd[b], srr[b])

  def wait_chunk(b):
    pltpu.make_async_copy(srcsc_hbm.at[s, pl.ds(0, CH)], rs[b], srr[b]).wait()
    pltpu.make_async_copy(dstsc_hbm.at[s, pl.ds(0, CH)], rd[b], srr[b]).wait()

  def den_wait():
    pltpu.make_async_copy(exr_v.at[0], den_sh.at[rd0.at[0]], sd).wait()

  def sstep(sref, dref, bb, dwait):
    jm = bb % 4
    for g in range(SUB // 16):
      sl = pl.ds(g * 16, 16)
      s16 = sref[bb, sl]
      d16 = dref[bb, sl]
      e = plsc.load_gather(as_v, [s16]) + plsc.load_gather(ad_v, [d16])
      e = jnp.maximum(e, 0.2 * e)
      exr_v[jm, sl] = jnp.exp(e)
    if dwait:
      den_wait()
    pltpu.async_copy(exr_v.at[jm], den_sh.at[dref.at[bb]], sd, add=True)

  stage_chunk(0, 0)
  stage_chunk(1, 1)
  wait_chunk(0)
  for bb in range(CH):
    sstep(rs0, rd0, bb, bb >= 2)
  stage_chunk(2, 0)

  def cpair(p, _):
    q1 = 2 * p + 1
    wait_chunk(1)
    for bb in range(CH):
      sstep(rs1, rd1, bb, True)
    pl.when(q1 + 2 < NCHK)(lambda: stage_chunk(q1 + 2, 1))
    q2 = 2 * p + 2
    wait_chunk(0)
    for bb in range(CH):
      sstep(rs0, rd0, bb, True)
    pl.when(q2 + 2 < NCHK)(lambda: stage_chunk(q2 + 2, 0))
    return 0

  lax.fori_loop(0, (NCHK - 1) // 2, cpair, 0)
  for bb in range(2):
    sstep(tsrc_v, tdst_v, bb, True)
  den_wait()
  den_wait()

  plsc.subcore_barrier()
  pltpu.sync_copy(den_sh.at[pl.ds(0, N)], denf_v)

  def group_a(j, sl):
    s16 = srow_v[j, sl]
    d16 = drow_v[j, sl]
    e = plsc.load_gather(as_v, [s16]) + plsc.load_gather(ad_v, [d16])
    e = jnp.maximum(e, 0.2 * e)
    den16 = plsc.load_gather(denf_v, [d16]) + 1e-16
    al = jnp.exp(e) / den16
    alr_v[lax.rem(j, 4), sl] = al
    return al

  def alpha_issue(j):
    pltpu.async_copy(alr_v.at[lax.rem(j, 4)], alpha_out.at[s, c, j], sal)

  def alpha_wait():
    pltpu.make_async_copy(alr_v.at[0], alpha_out.at[s, c, 0], sal).wait()

  _pipelined_row_pass(h2_hbm, srow_v, drow_v, num_sh, [g0, g1], [s0, s1],
                      [sg0, sg1], [ss0, ss1], SH, D2,
                      group_a, alpha_issue, alpha_wait)
  plsc.subcore_barrier()
  pltpu.sync_copy(num_sh.at[pl.ds(s * STR, STR), :],
                  num_out.at[c, pl.ds(s * STR, STR), :])


def kernel(x, edge_index, edge_weight, W1, att_src1, att_dst1, bias1,
           W2, att_src2, att_dst2, bias2):
  del edge_weight
  src = edge_index[0].astype(jnp.int32)
  dst = edge_index[1].astype(jnp.int32)
  src1 = src.reshape(NW, S1, SUB)
  dst1 = dst.reshape(NW, S1, SUB)
  src2 = src.reshape(NS, S2, SUB)
  dst2 = dst.reshape(NS, S2, SUB)

  att1 = jnp.stack([att_src1, att_dst1], axis=1)
  att2 = jnp.stack([att_src2, att_dst2], axis=1)

  h1, aux1 = pl.pallas_call(
      _tc1_body,
      out_shape=(jax.ShapeDtypeStruct((N, D1), jnp.float32),
                 jax.ShapeDtypeStruct((2, 1, N), jnp.float32)),
  )(x, W1, att1)

  num1p, den1p = _sc_layer1(src1, dst1, h1, aux1)

  h2, aux2 = pl.pallas_call(
      _tc2_body,
      out_shape=(jax.ShapeDtypeStruct((N, D2), jnp.float32),
                 jax.ShapeDtypeStruct((2, 1, N), jnp.float32)),
  )(num1p, den1p, bias1.reshape(1, D1), W2, att2)

  num2p, alpha4 = _sc_layer2(src2, dst2, h2, aux2)

  logp = pl.pallas_call(
      _tc3_body,
      out_shape=jax.ShapeDtypeStruct((N, D2), jnp.float32),
  )(num2p, bias2.reshape(1, D2))

  alpha = alpha4.reshape(E, 1)

  return (logp, edge_index, alpha)
